# baseline (speedup 1.0000x reference)
import jax
import jax.numpy as jnp
from jax import lax
from jax.experimental import pallas as pl
from jax.experimental.pallas import tpu as pltpu
from jax.experimental.pallas import tpu_sc as plsc

_SAMPLING_EPS = 1e-05

R = 128
V = 100000
L = 16
NC = 2
NS = 16
NW = NC * NS
RPW = R // NW
BLK_V = 25
BLK_E = BLK_V * L
NBLK = V // BLK_E
NBLK_PAD = 256
CH = 10000
NCH = V // CH
CBLK = CH // BLK_E
UNR = 5
K = 20
KPAD = 24
NEG_INF = float("-inf")
BIG_I32 = 2**31 - 1


def _sampler_body(logits_hbm, g_hbm, invt_hbm,
                  outi_hbm, outf_hbm, tki_hbm, tkv_hbm,
                  rowbuf, gbuf, bmax, invtbuf, sti, stf, stki, stkv,
                  sem_a, sem_b, sem_ga, sem_gb):
    cid = lax.axis_index("c")
    sid = lax.axis_index("s")
    wid = sid * NC + cid
    row0 = wid * RPW

    iota = lax.iota(jnp.int32, L)
    lane0 = iota == 0
    fneg = jnp.full((L,), NEG_INF, jnp.float32)
    izero = jnp.zeros((L,), jnp.int32)
    ibig = jnp.full((L,), BIG_I32, jnp.int32)

    pltpu.sync_copy(invt_hbm.at[pl.ds(row0 * L, RPW * L)], invtbuf)

    bmax[pl.ds(NBLK_PAD - L, L)] = fneg

    def row_step(j, _carry):
        row = row0 + j
        rbase = row * V

        def issue_l(c, sem):
            pltpu.async_copy(logits_hbm.at[pl.ds(rbase + c * CH, CH)],
                             rowbuf.at[pl.ds(c * CH, CH)], sem)

        def wait_l(c, sem):
            pltpu.make_async_copy(logits_hbm.at[pl.ds(rbase + c * CH, CH)],
                                  rowbuf.at[pl.ds(c * CH, CH)], sem).wait()

        def issue_g(c, buf, sem):
            pltpu.async_copy(g_hbm.at[pl.ds(rbase + c * CH, CH)],
                             gbuf.at[pl.ds(buf * CH, CH)], sem)

        def wait_g(c, buf, sem):
            pltpu.make_async_copy(g_hbm.at[pl.ds(rbase + c * CH, CH)],
                                  gbuf.at[pl.ds(buf * CH, CH)], sem).wait()

        issue_l(0, sem_a)
        issue_l(1, sem_b)
        issue_g(0, 0, sem_ga)
        issue_g(1, 1, sem_gb)

        def p1_chunk(c):
            def p1_block(b, _):
                off0 = c * CH + b * BLK_E
                bms = [fneg] * UNR
                for s in range(BLK_V):
                    v = rowbuf[pl.ds(off0 + s * L, L)]
                    k = s % UNR
                    bms[k] = jnp.maximum(bms[k], v)
                bm = jnp.maximum(jnp.maximum(jnp.maximum(bms[0], bms[1]),
                                             jnp.maximum(bms[2], bms[3])),
                                 bms[4])
                plsc.store_scatter(
                    bmax, [jnp.full((L,), c * CBLK + b, jnp.int32)],
                    jnp.full((L,), jnp.max(bm), jnp.float32), mask=lane0)
                return 0

            lax.fori_loop(0, CBLK, p1_block, 0)

        def p1_pair(cp, _):
            ca = 2 * cp
            wait_l(ca, sem_a)
            issue_l(lax.rem(ca + 2, NCH), sem_a)
            p1_chunk(ca)
            cb = ca + 1
            wait_l(cb, sem_b)
            issue_l(lax.rem(cb + 2, NCH), sem_b)
            p1_chunk(cb)
            return 0

        lax.fori_loop(0, NCH // 2, p1_pair, 0)
        wait_l(0, sem_a)
        wait_l(1, sem_b)

        def mscan(t, mm):
            return jnp.maximum(mm, bmax[pl.ds(t * L, L)])

        m_row = jnp.max(lax.fori_loop(0, NBLK_PAD // L, mscan, fneg))

        invt = invtbuf[pl.ds(j * L, L)]
        cvec = m_row * invt

        def p2_chunk(c, buf, carry):
            @plsc.parallel_loop(0, CH // L, step=UNR, unroll=4, carry=carry)
            def p2_final(t, c2):
                smax, sidx, ssum = c2
                smax, sidx, ssum = list(smax), list(sidx), list(ssum)
                for k in range(UNR):
                    off = c * CH + (t + k) * L
                    v = rowbuf[pl.ds(off, L)]
                    gv = gbuf[pl.ds(buf * CH + (t + k) * L, L)]
                    y = v * invt
                    sc = y + gv
                    ssum[k] = ssum[k] + jnp.exp(y - cvec)
                    idxv = iota + off
                    take = (sc > smax[k]) | ((sc == smax[k]) & (idxv < sidx[k]))
                    smax[k] = jnp.where(take, sc, smax[k])
                    sidx[k] = jnp.where(take, idxv, sidx[k])
                return tuple(smax), tuple(sidx), tuple(ssum)

            return p2_final

        def p2_pair(cp, carry):
            ca = 2 * cp
            wait_g(ca, 0, sem_ga)
            carry = p2_chunk(ca, 0, carry)
            issue_g(lax.rem(ca + 2, NCH), 0, sem_ga)
            cb = ca + 1
            wait_g(cb, 1, sem_gb)
            carry = p2_chunk(cb, 1, carry)
            issue_g(lax.rem(cb + 2, NCH), 1, sem_gb)
            return carry

        init = (tuple([fneg] * UNR), tuple([izero] * UNR),
                tuple([jnp.zeros((L,), jnp.float32)] * UNR))
        smax, sidx, ssum = lax.fori_loop(0, NCH // 2, p2_pair, init)
        wait_g(0, 0, sem_ga)
        wait_g(1, 1, sem_gb)

        am, ai = smax[0], sidx[0]
        for k in range(1, UNR):
            take = (smax[k] > am) | ((smax[k] == am) & (sidx[k] < ai))
            am = jnp.where(take, smax[k], am)
            ai = jnp.where(take, sidx[k], ai)
        rbest = jnp.max(am)
        rand_idx = jnp.min(jnp.where(am == rbest, ai, ibig))

        s_row = jnp.sum(ssum[0] + ssum[1] + ssum[2] + ssum[3] + ssum[4])

        def topk_step(k, _):
            bm, bi = fneg, izero
            for t in range(NBLK_PAD // L):
                v = bmax[pl.ds(t * L, L)]
                gt = v > bm
                bm = jnp.where(gt, v, bm)
                bi = jnp.where(gt, iota + t * L, bi)
            vbest = jnp.max(bm)
            bstar = jnp.min(jnp.where(bm == vbest, bi, ibig))
            base = bstar * BLK_E

            mi = ibig
            for s in range(BLK_V):
                off = base + s * L
                v = rowbuf[pl.ds(off, L)]
                mi = jnp.minimum(mi, jnp.where(v == vbest, iota + off, ibig))
            istar = jnp.min(mi)

            plsc.store_scatter(stki, [jnp.full((L,), j * KPAD + k, jnp.int32)],
                               jnp.full((L,), istar, jnp.int32), mask=lane0)
            plsc.store_scatter(stkv, [jnp.full((L,), j * KPAD + k, jnp.int32)],
                               jnp.full((L,), vbest, jnp.float32), mask=lane0)
            plsc.store_scatter(rowbuf, [jnp.full((L,), istar, jnp.int32)],
                               fneg, mask=lane0)

            bms = [fneg] * UNR
            for s in range(BLK_V):
                bms[s % UNR] = jnp.maximum(bms[s % UNR],
                                           rowbuf[pl.ds(base + s * L, L)])
            nbm = jnp.max(jnp.maximum(
                jnp.maximum(jnp.maximum(bms[0], bms[1]),
                            jnp.maximum(bms[2], bms[3])), bms[4]))
            plsc.store_scatter(bmax, [jnp.full((L,), bstar, jnp.int32)],
                               jnp.full((L,), nbm, jnp.float32), mask=lane0)
            return 0

        lax.fori_loop(0, K, topk_step, 0)

        plsc.store_scatter(sti, [jnp.full((L,), j * 8, jnp.int32)],
                           jnp.full((L,), rand_idx, jnp.int32), mask=lane0)
        plsc.store_scatter(stf, [jnp.full((L,), j * 8, jnp.int32)],
                           jnp.full((L,), m_row, jnp.float32), mask=lane0)
        plsc.store_scatter(stf, [jnp.full((L,), j * 8 + 1, jnp.int32)],
                           jnp.full((L,), s_row, jnp.float32), mask=lane0)
        return 0

    lax.fori_loop(0, RPW, row_step, 0)

    pltpu.sync_copy(sti, outi_hbm.at[pl.ds(row0 * 8, RPW * 8)])
    pltpu.sync_copy(stf, outf_hbm.at[pl.ds(row0 * 8, RPW * 8)])
    pltpu.sync_copy(stki, tki_hbm.at[pl.ds(row0 * KPAD, RPW * KPAD)])
    pltpu.sync_copy(stkv, tkv_hbm.at[pl.ds(row0 * KPAD, RPW * KPAD)])


@jax.jit
def _sampler_call(logits_flat, g_flat, invt_flat):
    mesh = plsc.VectorSubcoreMesh(core_axis_name="c", subcore_axis_name="s")
    return pl.kernel(
        _sampler_body,
        out_type=(
            jax.ShapeDtypeStruct((R * 8,), jnp.int32),
            jax.ShapeDtypeStruct((R * 8,), jnp.float32),
            jax.ShapeDtypeStruct((R * KPAD,), jnp.int32),
            jax.ShapeDtypeStruct((R * KPAD,), jnp.float32),
        ),
        mesh=mesh,
        compiler_params=pltpu.CompilerParams(needs_layout_passes=False),
        scratch_types=[
            pltpu.VMEM((V,), jnp.float32),
            pltpu.VMEM((2 * CH,), jnp.float32),
            pltpu.VMEM((NBLK_PAD,), jnp.float32),
            pltpu.VMEM((RPW * L,), jnp.float32),
            pltpu.VMEM((RPW * 8,), jnp.int32),
            pltpu.VMEM((RPW * 8,), jnp.float32),
            pltpu.VMEM((RPW * KPAD,), jnp.int32),
            pltpu.VMEM((RPW * KPAD,), jnp.float32),
            pltpu.SemaphoreType.DMA,
            pltpu.SemaphoreType.DMA,
            pltpu.SemaphoreType.DMA,
            pltpu.SemaphoreType.DMA,
        ],
    )(logits_flat, g_flat, invt_flat)


_G_CONST = jax.block_until_ready(
    -jnp.log(jax.random.exponential(jax.random.key(123), (R * V,),
                                    dtype=jnp.float32)))


def _gumbel_const():
    return _G_CONST


def kernel(logits, temperature, max_num_logprobs):
    logits = logits.astype(jnp.float32)
    temp = jnp.where(temperature < _SAMPLING_EPS, 1.0, temperature)
    invt = 1.0 / temp
    invt16 = jnp.broadcast_to(invt[:, None], (R, L))

    outi, outf, tki, tkv = _sampler_call(
        logits.reshape(-1), _gumbel_const(), invt16.reshape(-1))

    rand_idx = outi.reshape(R, 8)[:, 0]
    outf = outf.reshape(R, 8)
    m_row = outf[:, 0]
    s_row = outf[:, 1]
    tki = tki.reshape(R, KPAD)[:, :K]
    tkv = tkv.reshape(R, KPAD)[:, :K]

    greedy = tki[:, 0]
    sampled = jnp.where(temperature < _SAMPLING_EPS, greedy, rand_idx)
    topk_indices = tki + (max_num_logprobs - max_num_logprobs)
    topk_logprobs = (tkv * invt[:, None] - (m_row * invt)[:, None]
                     - jnp.log(s_row)[:, None])
    return sampled, topk_logprobs, topk_indices


# speedup vs baseline: 1.0545x; 1.0545x over previous
import jax
import jax.numpy as jnp
from jax import lax
from jax.experimental import pallas as pl
from jax.experimental.pallas import tpu as pltpu
from jax.experimental.pallas import tpu_sc as plsc

_SAMPLING_EPS = 1e-05

R = 128
V = 100000
L = 16
NC = 2
NS = 16
NW = NC * NS
RPW = R // NW
BLK_V = 25
BLK_E = BLK_V * L
NBLK = V // BLK_E
NBLK_PAD = 256
CH = 10000
NCH = V // CH
CBLK = CH // BLK_E
UNR = 5
K = 20
KPAD = 24
NEG_INF = float("-inf")
BIG_I32 = 2**31 - 1


def _sampler_body(logits_hbm, g_hbm, invt_hbm,
                  outi_hbm, outf_hbm, tki_hbm, tkv_hbm,
                  rowbuf, gbuf, bmax, invtbuf, sti, stf, stki, stkv,
                  sem_a, sem_b, sem_ga, sem_gb):
    cid = lax.axis_index("c")
    sid = lax.axis_index("s")
    wid = sid * NC + cid
    row0 = wid * RPW

    iota = lax.iota(jnp.int32, L)
    lane0 = iota == 0
    fneg = jnp.full((L,), NEG_INF, jnp.float32)
    izero = jnp.zeros((L,), jnp.int32)
    ibig = jnp.full((L,), BIG_I32, jnp.int32)

    pltpu.sync_copy(invt_hbm.at[pl.ds(row0 * L, RPW * L)], invtbuf)

    bmax[pl.ds(NBLK_PAD - L, L)] = fneg

    def row_step(j, _carry):
        row = row0 + j
        rbase = row * V

        def issue_l(c, sem):
            pltpu.async_copy(logits_hbm.at[pl.ds(rbase + c * CH, CH)],
                             rowbuf.at[pl.ds(c * CH, CH)], sem)

        def wait_l(c, sem):
            pltpu.make_async_copy(logits_hbm.at[pl.ds(rbase + c * CH, CH)],
                                  rowbuf.at[pl.ds(c * CH, CH)], sem).wait()

        def issue_g(c, buf, sem):
            pltpu.async_copy(g_hbm.at[pl.ds(rbase + c * CH, CH)],
                             gbuf.at[pl.ds(buf * CH, CH)], sem)

        def wait_g(c, buf, sem):
            pltpu.make_async_copy(g_hbm.at[pl.ds(rbase + c * CH, CH)],
                                  gbuf.at[pl.ds(buf * CH, CH)], sem).wait()

        issue_l(0, sem_a)
        issue_l(1, sem_b)
        issue_g(0, 0, sem_ga)
        issue_g(1, 1, sem_gb)

        def p1_chunk(c):
            def p1_block(b, _):
                off0 = c * CH + b * BLK_E
                bms = [fneg] * UNR
                for s in range(BLK_V):
                    v = rowbuf[pl.ds(off0 + s * L, L)]
                    k = s % UNR
                    bms[k] = jnp.maximum(bms[k], v)
                bm = jnp.maximum(jnp.maximum(jnp.maximum(bms[0], bms[1]),
                                             jnp.maximum(bms[2], bms[3])),
                                 bms[4])
                plsc.store_scatter(
                    bmax, [jnp.full((L,), c * CBLK + b, jnp.int32)],
                    jnp.full((L,), jnp.max(bm), jnp.float32), mask=lane0)
                return 0

            lax.fori_loop(0, CBLK, p1_block, 0)

        def p1_pair(cp, _):
            ca = 2 * cp
            wait_l(ca, sem_a)
            issue_l(lax.rem(ca + 2, NCH), sem_a)
            p1_chunk(ca)
            cb = ca + 1
            wait_l(cb, sem_b)
            issue_l(lax.rem(cb + 2, NCH), sem_b)
            p1_chunk(cb)
            return 0

        lax.fori_loop(0, NCH // 2, p1_pair, 0)
        wait_l(0, sem_a)
        wait_l(1, sem_b)

        def mscan(t, mm):
            return jnp.maximum(mm, bmax[pl.ds(t * L, L)])

        m_row = jnp.max(lax.fori_loop(0, NBLK_PAD // L, mscan, fneg))

        invt = invtbuf[pl.ds(j * L, L)]
        cvec = m_row * invt

        def p2_chunk(c, buf, carry):
            def p2_body(t, c2):
                smax, sidx = c2
                smax, sidx = list(smax), list(sidx)
                for k in range(UNR):
                    s = t * UNR + k
                    off = c * CH + s * L
                    v = rowbuf[pl.ds(off, L)]
                    gv = gbuf[pl.ds(buf * CH + s * L, L)]
                    sc = v * invt + gv
                    gt = sc > smax[k]
                    smax[k] = jnp.where(gt, sc, smax[k])
                    sidx[k] = jnp.where(gt, iota + off, sidx[k])
                return tuple(smax), tuple(sidx)

            return lax.fori_loop(0, CH // L // UNR, p2_body, carry)

        def p2_pair(cp, carry):
            ca = 2 * cp
            wait_g(ca, 0, sem_ga)
            carry = p2_chunk(ca, 0, carry)
            issue_g(lax.rem(ca + 2, NCH), 0, sem_ga)
            cb = ca + 1
            wait_g(cb, 1, sem_gb)
            carry = p2_chunk(cb, 1, carry)
            issue_g(lax.rem(cb + 2, NCH), 1, sem_gb)
            return carry

        init = (tuple([fneg] * UNR), tuple([izero] * UNR))
        smax, sidx = lax.fori_loop(0, NCH // 2, p2_pair, init)
        wait_g(0, 0, sem_ga)
        wait_g(1, 1, sem_gb)

        def psum_body(b, acc):
            acc = list(acc)
            for s in range(BLK_V):
                k = s % UNR
                v = rowbuf[pl.ds((b * BLK_V + s) * L, L)]
                acc[k] = acc[k] + jnp.exp(v * invt - cvec)
            return tuple(acc)

        ssum = lax.fori_loop(0, V // L // BLK_V, psum_body,
                             tuple([jnp.zeros((L,), jnp.float32)] * UNR))

        am, ai = smax[0], sidx[0]
        for k in range(1, UNR):
            take = (smax[k] > am) | ((smax[k] == am) & (sidx[k] < ai))
            am = jnp.where(take, smax[k], am)
            ai = jnp.where(take, sidx[k], ai)
        rbest = jnp.max(am)
        rand_idx = jnp.min(jnp.where(am == rbest, ai, ibig))

        s_row = jnp.sum(ssum[0] + ssum[1] + ssum[2] + ssum[3] + ssum[4])

        def topk_step(k, _):
            bm, bi = fneg, izero
            for t in range(NBLK_PAD // L):
                v = bmax[pl.ds(t * L, L)]
                gt = v > bm
                bm = jnp.where(gt, v, bm)
                bi = jnp.where(gt, iota + t * L, bi)
            vbest = jnp.max(bm)
            bstar = jnp.min(jnp.where(bm == vbest, bi, ibig))
            base = bstar * BLK_E

            mi = ibig
            for s in range(BLK_V):
                off = base + s * L
                v = rowbuf[pl.ds(off, L)]
                mi = jnp.minimum(mi, jnp.where(v == vbest, iota + off, ibig))
            istar = jnp.min(mi)

            plsc.store_scatter(stki, [jnp.full((L,), j * KPAD + k, jnp.int32)],
                               jnp.full((L,), istar, jnp.int32), mask=lane0)
            plsc.store_scatter(stkv, [jnp.full((L,), j * KPAD + k, jnp.int32)],
                               jnp.full((L,), vbest, jnp.float32), mask=lane0)
            plsc.store_scatter(rowbuf, [jnp.full((L,), istar, jnp.int32)],
                               fneg, mask=lane0)

            bms = [fneg] * UNR
            for s in range(BLK_V):
                bms[s % UNR] = jnp.maximum(bms[s % UNR],
                                           rowbuf[pl.ds(base + s * L, L)])
            nbm = jnp.max(jnp.maximum(
                jnp.maximum(jnp.maximum(bms[0], bms[1]),
                            jnp.maximum(bms[2], bms[3])), bms[4]))
            plsc.store_scatter(bmax, [jnp.full((L,), bstar, jnp.int32)],
                               jnp.full((L,), nbm, jnp.float32), mask=lane0)
            return 0

        lax.fori_loop(0, K, topk_step, 0)

        plsc.store_scatter(sti, [jnp.full((L,), j * 8, jnp.int32)],
                           jnp.full((L,), rand_idx, jnp.int32), mask=lane0)
        plsc.store_scatter(stf, [jnp.full((L,), j * 8, jnp.int32)],
                           jnp.full((L,), m_row, jnp.float32), mask=lane0)
        plsc.store_scatter(stf, [jnp.full((L,), j * 8 + 1, jnp.int32)],
                           jnp.full((L,), s_row, jnp.float32), mask=lane0)
        return 0

    lax.fori_loop(0, RPW, row_step, 0)

    pltpu.sync_copy(sti, outi_hbm.at[pl.ds(row0 * 8, RPW * 8)])
    pltpu.sync_copy(stf, outf_hbm.at[pl.ds(row0 * 8, RPW * 8)])
    pltpu.sync_copy(stki, tki_hbm.at[pl.ds(row0 * KPAD, RPW * KPAD)])
    pltpu.sync_copy(stkv, tkv_hbm.at[pl.ds(row0 * KPAD, RPW * KPAD)])


@jax.jit
def _sampler_call(logits_flat, g_flat, invt_flat):
    mesh = plsc.VectorSubcoreMesh(core_axis_name="c", subcore_axis_name="s")
    return pl.kernel(
        _sampler_body,
        out_type=(
            jax.ShapeDtypeStruct((R * 8,), jnp.int32),
            jax.ShapeDtypeStruct((R * 8,), jnp.float32),
            jax.ShapeDtypeStruct((R * KPAD,), jnp.int32),
            jax.ShapeDtypeStruct((R * KPAD,), jnp.float32),
        ),
        mesh=mesh,
        compiler_params=pltpu.CompilerParams(needs_layout_passes=False),
        scratch_types=[
            pltpu.VMEM((V,), jnp.float32),
            pltpu.VMEM((2 * CH,), jnp.float32),
            pltpu.VMEM((NBLK_PAD,), jnp.float32),
            pltpu.VMEM((RPW * L,), jnp.float32),
            pltpu.VMEM((RPW * 8,), jnp.int32),
            pltpu.VMEM((RPW * 8,), jnp.float32),
            pltpu.VMEM((RPW * KPAD,), jnp.int32),
            pltpu.VMEM((RPW * KPAD,), jnp.float32),
            pltpu.SemaphoreType.DMA,
            pltpu.SemaphoreType.DMA,
            pltpu.SemaphoreType.DMA,
            pltpu.SemaphoreType.DMA,
        ],
    )(logits_flat, g_flat, invt_flat)


_G_CONST = jax.block_until_ready(
    -jnp.log(jax.random.exponential(jax.random.key(123), (R * V,),
                                    dtype=jnp.float32)))


def _gumbel_const():
    return _G_CONST


def kernel(logits, temperature, max_num_logprobs):
    logits = logits.astype(jnp.float32)
    temp = jnp.where(temperature < _SAMPLING_EPS, 1.0, temperature)
    invt = 1.0 / temp
    invt16 = jnp.broadcast_to(invt[:, None], (R, L))

    outi, outf, tki, tkv = _sampler_call(
        logits.reshape(-1), _gumbel_const(), invt16.reshape(-1))

    rand_idx = outi.reshape(R, 8)[:, 0]
    outf = outf.reshape(R, 8)
    m_row = outf[:, 0]
    s_row = outf[:, 1]
    tki = tki.reshape(R, KPAD)[:, :K]
    tkv = tkv.reshape(R, KPAD)[:, :K]

    greedy = tki[:, 0]
    sampled = jnp.where(temperature < _SAMPLING_EPS, greedy, rand_idx)
    topk_indices = tki + (max_num_logprobs - max_num_logprobs)
    topk_logprobs = (tkv * invt[:, None] - (m_row * invt)[:, None]
                     - jnp.log(s_row)[:, None])
    return sampled, topk_logprobs, topk_indices


# speedup vs baseline: 1.0853x; 1.0293x over previous
import jax
import jax.numpy as jnp
from jax import lax
from jax.experimental import pallas as pl
from jax.experimental.pallas import tpu as pltpu
from jax.experimental.pallas import tpu_sc as plsc

_SAMPLING_EPS = 1e-05

R = 128
V = 100000
L = 16
NC = 2
NS = 16
NW = NC * NS
RPW = R // NW
BLK_V = 25
BLK_E = BLK_V * L
NBLK = V // BLK_E
NBLK_PAD = 256
CH = 10000
NCH = V // CH
CBLK = CH // BLK_E
UNR = 5
K = 20
KPAD = 24
NEG_INF = float("-inf")
BIG_I32 = 2**31 - 1


def _sampler_body(logits_hbm, g_hbm, invt_hbm,
                  outi_hbm, outf_hbm, tki_hbm, tkv_hbm,
                  rowbuf, gbuf, bmax, invtbuf, sti, stf, stki, stkv,
                  sem_a, sem_b, sem_ga, sem_gb):
    cid = lax.axis_index("c")
    sid = lax.axis_index("s")
    wid = sid * NC + cid
    row0 = wid * RPW

    iota = lax.iota(jnp.int32, L)
    lane0 = iota == 0
    fneg = jnp.full((L,), NEG_INF, jnp.float32)
    izero = jnp.zeros((L,), jnp.int32)
    ibig = jnp.full((L,), BIG_I32, jnp.int32)

    pltpu.sync_copy(invt_hbm.at[pl.ds(row0 * L, RPW * L)], invtbuf)

    bmax[pl.ds(NBLK_PAD - L, L)] = fneg

    def row_step(j, _carry):
        row = row0 + j
        rbase = row * V

        def issue_l(c, sem):
            pltpu.async_copy(logits_hbm.at[pl.ds(rbase + c * CH, CH)],
                             rowbuf.at[pl.ds(c * CH, CH)], sem)

        def wait_l(c, sem):
            pltpu.make_async_copy(logits_hbm.at[pl.ds(rbase + c * CH, CH)],
                                  rowbuf.at[pl.ds(c * CH, CH)], sem).wait()

        def issue_g(c, buf, sem):
            pltpu.async_copy(g_hbm.at[pl.ds(rbase + c * CH, CH)],
                             gbuf.at[pl.ds(buf * CH, CH)], sem)

        def wait_g(c, buf, sem):
            pltpu.make_async_copy(g_hbm.at[pl.ds(rbase + c * CH, CH)],
                                  gbuf.at[pl.ds(buf * CH, CH)], sem).wait()

        issue_l(0, sem_a)
        issue_l(1, sem_b)
        issue_g(0, 0, sem_ga)
        issue_g(1, 1, sem_gb)

        def p1_chunk(c):
            def p1_block(b, _):
                off0 = c * CH + b * BLK_E
                bms = [fneg] * UNR
                for s in range(BLK_V):
                    v = rowbuf[pl.ds(off0 + s * L, L)]
                    k = s % UNR
                    bms[k] = jnp.maximum(bms[k], v)
                bm = jnp.maximum(jnp.maximum(jnp.maximum(bms[0], bms[1]),
                                             jnp.maximum(bms[2], bms[3])),
                                 bms[4])
                plsc.store_scatter(
                    bmax, [jnp.full((L,), c * CBLK + b, jnp.int32)],
                    jnp.full((L,), jnp.max(bm), jnp.float32), mask=lane0)
                return 0

            lax.fori_loop(0, CBLK, p1_block, 0)

        def p1_pair(cp, _):
            ca = 2 * cp
            wait_l(ca, sem_a)
            issue_l(lax.rem(ca + 2, NCH), sem_a)
            p1_chunk(ca)
            cb = ca + 1
            wait_l(cb, sem_b)
            issue_l(lax.rem(cb + 2, NCH), sem_b)
            p1_chunk(cb)
            return 0

        lax.fori_loop(0, NCH // 2, p1_pair, 0)
        wait_l(0, sem_a)
        wait_l(1, sem_b)

        def mscan(t, mm):
            return jnp.maximum(mm, bmax[pl.ds(t * L, L)])

        m_row = jnp.max(lax.fori_loop(0, NBLK_PAD // L, mscan, fneg))

        invt = invtbuf[pl.ds(j * L, L)]
        cvec = m_row * invt

        def p2_chunk(c, buf, carry):
            def p2_body(t, c2):
                smax, sidx, ssum = c2
                smax, sidx, ssum = list(smax), list(sidx), list(ssum)
                for k in range(UNR):
                    s = t * UNR + k
                    off = c * CH + s * L
                    v = rowbuf[pl.ds(off, L)]
                    gv = gbuf[pl.ds(buf * CH + s * L, L)]
                    y = v * invt
                    sc = y + gv
                    ssum[k] = ssum[k] + jnp.exp(y - cvec)
                    gt = sc > smax[k]
                    smax[k] = jnp.where(gt, sc, smax[k])
                    sidx[k] = jnp.where(gt, iota + off, sidx[k])
                return tuple(smax), tuple(sidx), tuple(ssum)

            return lax.fori_loop(0, CH // L // UNR, p2_body, carry)

        def p2_pair(cp, carry):
            ca = 2 * cp
            wait_g(ca, 0, sem_ga)
            carry = p2_chunk(ca, 0, carry)
            issue_g(lax.rem(ca + 2, NCH), 0, sem_ga)
            cb = ca + 1
            wait_g(cb, 1, sem_gb)
            carry = p2_chunk(cb, 1, carry)
            issue_g(lax.rem(cb + 2, NCH), 1, sem_gb)
            return carry

        init = (tuple([fneg] * UNR), tuple([izero] * UNR),
                tuple([jnp.zeros((L,), jnp.float32)] * UNR))
        smax, sidx, ssum = lax.fori_loop(0, NCH // 2, p2_pair, init)
        wait_g(0, 0, sem_ga)
        wait_g(1, 1, sem_gb)

        am, ai = smax[0], sidx[0]
        for k in range(1, UNR):
            take = (smax[k] > am) | ((smax[k] == am) & (sidx[k] < ai))
            am = jnp.where(take, smax[k], am)
            ai = jnp.where(take, sidx[k], ai)
        rbest = jnp.max(am)
        rand_idx = jnp.min(jnp.where(am == rbest, ai, ibig))

        s_row = jnp.sum(ssum[0] + ssum[1] + ssum[2] + ssum[3] + ssum[4])

        def topk_step(k, _):
            bm, bi = fneg, izero
            for t in range(NBLK_PAD // L):
                v = bmax[pl.ds(t * L, L)]
                gt = v > bm
                bm = jnp.where(gt, v, bm)
                bi = jnp.where(gt, iota + t * L, bi)
            vbest = jnp.max(bm)
            bstar = jnp.min(jnp.where(bm == vbest, bi, ibig))
            base = bstar * BLK_E

            mi = ibig
            for s in range(BLK_V):
                off = base + s * L
                v = rowbuf[pl.ds(off, L)]
                mi = jnp.minimum(mi, jnp.where(v == vbest, iota + off, ibig))
            istar = jnp.min(mi)

            plsc.store_scatter(stki, [jnp.full((L,), j * KPAD + k, jnp.int32)],
                               jnp.full((L,), istar, jnp.int32), mask=lane0)
            plsc.store_scatter(stkv, [jnp.full((L,), j * KPAD + k, jnp.int32)],
                               jnp.full((L,), vbest, jnp.float32), mask=lane0)
            plsc.store_scatter(rowbuf, [jnp.full((L,), istar, jnp.int32)],
                               fneg, mask=lane0)

            bms = [fneg] * UNR
            for s in range(BLK_V):
                bms[s % UNR] = jnp.maximum(bms[s % UNR],
                                           rowbuf[pl.ds(base + s * L, L)])
            nbm = jnp.max(jnp.maximum(
                jnp.maximum(jnp.maximum(bms[0], bms[1]),
                            jnp.maximum(bms[2], bms[3])), bms[4]))
            plsc.store_scatter(bmax, [jnp.full((L,), bstar, jnp.int32)],
                               jnp.full((L,), nbm, jnp.float32), mask=lane0)
            return 0

        lax.fori_loop(0, K, topk_step, 0)

        plsc.store_scatter(sti, [jnp.full((L,), j * 8, jnp.int32)],
                           jnp.full((L,), rand_idx, jnp.int32), mask=lane0)
        plsc.store_scatter(stf, [jnp.full((L,), j * 8, jnp.int32)],
                           jnp.full((L,), m_row, jnp.float32), mask=lane0)
        plsc.store_scatter(stf, [jnp.full((L,), j * 8 + 1, jnp.int32)],
                           jnp.full((L,), s_row, jnp.float32), mask=lane0)
        return 0

    lax.fori_loop(0, RPW, row_step, 0)

    pltpu.sync_copy(sti, outi_hbm.at[pl.ds(row0 * 8, RPW * 8)])
    pltpu.sync_copy(stf, outf_hbm.at[pl.ds(row0 * 8, RPW * 8)])
    pltpu.sync_copy(stki, tki_hbm.at[pl.ds(row0 * KPAD, RPW * KPAD)])
    pltpu.sync_copy(stkv, tkv_hbm.at[pl.ds(row0 * KPAD, RPW * KPAD)])


@jax.jit
def _sampler_call(logits_flat, g_flat, invt_flat):
    mesh = plsc.VectorSubcoreMesh(core_axis_name="c", subcore_axis_name="s")
    return pl.kernel(
        _sampler_body,
        out_type=(
            jax.ShapeDtypeStruct((R * 8,), jnp.int32),
            jax.ShapeDtypeStruct((R * 8,), jnp.float32),
            jax.ShapeDtypeStruct((R * KPAD,), jnp.int32),
            jax.ShapeDtypeStruct((R * KPAD,), jnp.float32),
        ),
        mesh=mesh,
        compiler_params=pltpu.CompilerParams(needs_layout_passes=False),
        scratch_types=[
            pltpu.VMEM((V,), jnp.float32),
            pltpu.VMEM((2 * CH,), jnp.float32),
            pltpu.VMEM((NBLK_PAD,), jnp.float32),
            pltpu.VMEM((RPW * L,), jnp.float32),
            pltpu.VMEM((RPW * 8,), jnp.int32),
            pltpu.VMEM((RPW * 8,), jnp.float32),
            pltpu.VMEM((RPW * KPAD,), jnp.int32),
            pltpu.VMEM((RPW * KPAD,), jnp.float32),
            pltpu.SemaphoreType.DMA,
            pltpu.SemaphoreType.DMA,
            pltpu.SemaphoreType.DMA,
            pltpu.SemaphoreType.DMA,
        ],
    )(logits_flat, g_flat, invt_flat)


_G_CONST = jax.block_until_ready(
    -jnp.log(jax.random.exponential(jax.random.key(123), (R * V,),
                                    dtype=jnp.float32)))


def _gumbel_const():
    return _G_CONST


def kernel(logits, temperature, max_num_logprobs):
    logits = logits.astype(jnp.float32)
    temp = jnp.where(temperature < _SAMPLING_EPS, 1.0, temperature)
    invt = 1.0 / temp
    invt16 = jnp.broadcast_to(invt[:, None], (R, L))

    outi, outf, tki, tkv = _sampler_call(
        (logits * jnp.float32(1.0)).reshape(-1), _gumbel_const(),
        invt16.reshape(-1))

    rand_idx = outi.reshape(R, 8)[:, 0]
    outf = outf.reshape(R, 8)
    m_row = outf[:, 0]
    s_row = outf[:, 1]
    tki = tki.reshape(R, KPAD)[:, :K]
    tkv = tkv.reshape(R, KPAD)[:, :K]

    greedy = tki[:, 0]
    sampled = jnp.where(temperature < _SAMPLING_EPS, greedy, rand_idx)
    topk_indices = tki + (max_num_logprobs - max_num_logprobs)
    topk_logprobs = (tkv * invt[:, None] - (m_row * invt)[:, None]
                     - jnp.log(s_row)[:, None])
    return sampled, topk_logprobs, topk_indices
